# Initial kernel scaffold; baseline (speedup 1.0000x reference)
import jax
import jax.numpy as jnp
from jax.experimental import pallas as pl


def kernel(in_degree, out_degree, in_table, out_table):
    raise NotImplementedError("write your pallas kernel here")


# speedup vs baseline: 2.1324x; 2.1324x over previous
import functools

import jax
import jax.numpy as jnp
from jax import lax
from jax.experimental import pallas as pl
from jax.experimental.pallas import tpu as pltpu
from jax.experimental.pallas import tpu_sc as plsc

N = 50000
D = 256
MAX_DEGREE = 512
V = MAX_DEGREE + 1
NC = 2
NS = 16
NW = NC * NS
CHUNK = 112
ROWS_PER_W = 1568
NPAD = NW * ROWS_PER_W
CHUNKS_PER_W = ROWS_PER_W // CHUNK
TAIL_W = NW - 1
TAIL_CHUNKS = 13
TAIL_ROWS = N - (TAIL_W * ROWS_PER_W + (TAIL_CHUNKS - 1) * CHUNK)


def _clip_remap(idx_ref):
    def body(j, _):
        v = idx_ref[pl.ds(j * 16, 16)]
        v = jnp.minimum(jnp.maximum(v, 0), MAX_DEGREE)
        idx_ref[pl.ds(j * 16, 16)] = jnp.where(v == 0, V, v)
        return 0
    lax.fori_loop(0, CHUNK // 16, body, 0, unroll=True)


def _sc_body(ind_hbm, outd_hbm, itab_hbm, otab_hbm, out_hbm,
             idx_a, idx_b, rows_a, rows_b, sem_a, sem_b):
    wid = lax.axis_index("s") * NC + lax.axis_index("c")
    w0 = wid * ROWS_PER_W
    nchunks = jnp.where(wid == TAIL_W, TAIL_CHUNKS, CHUNKS_PER_W)

    def chunk_body(c, _):
        base = w0 + c * CHUNK
        pltpu.sync_copy(ind_hbm.at[pl.ds(base, CHUNK)], idx_a)
        pltpu.sync_copy(outd_hbm.at[pl.ds(base, CHUNK)], idx_b)
        _clip_remap(idx_a)
        _clip_remap(idx_b)
        cp_a = pltpu.async_copy(itab_hbm.at[idx_a], rows_a, sem_a)
        cp_b = pltpu.async_copy(otab_hbm.at[idx_b], rows_b, sem_b)
        cp_a.wait()
        cp_b.wait()

        def add_row(r, _):
            for j in range(D // 16):
                plsc.addupdate(rows_a.at[r, pl.ds(j * 16, 16)],
                               rows_b[r, pl.ds(j * 16, 16)])
            return 0
        lax.fori_loop(0, CHUNK, add_row, 0)

        is_partial = jnp.logical_and(wid == TAIL_W, c == TAIL_CHUNKS - 1)

        @pl.when(jnp.logical_not(is_partial))
        def _():
            pltpu.sync_copy(rows_a, out_hbm.at[pl.ds(base, CHUNK)])

        @pl.when(is_partial)
        def _():
            pltpu.sync_copy(rows_a.at[pl.ds(0, TAIL_ROWS)],
                            out_hbm.at[pl.ds(base, TAIL_ROWS)])
        return 0

    lax.fori_loop(0, nchunks, chunk_body, 0)


@functools.partial(jax.jit, donate_argnums=())
def _centrality(ind_p, outd_p, itab, otab):
    mesh = plsc.VectorSubcoreMesh(core_axis_name="c", subcore_axis_name="s",
                                  num_cores=NC, num_subcores=NS)
    return pl.kernel(
        _sc_body,
        out_type=jax.ShapeDtypeStruct((N, D), jnp.float32),
        mesh=mesh,
        scratch_types=[
            pltpu.VMEM((CHUNK,), jnp.int32),
            pltpu.VMEM((CHUNK,), jnp.int32),
            pltpu.VMEM((CHUNK, D), jnp.float32),
            pltpu.VMEM((CHUNK, D), jnp.float32),
            pltpu.SemaphoreType.DMA,
            pltpu.SemaphoreType.DMA,
        ],
    )(ind_p, outd_p, itab, otab)


def kernel(in_degree, out_degree, in_table, out_table):
    zero_row = jnp.zeros((1, D), jnp.float32)
    itab = jnp.concatenate([in_table, zero_row], axis=0)
    otab = jnp.concatenate([out_table, zero_row], axis=0)
    pad = jnp.zeros((NPAD - N,), jnp.int32)
    ind_p = jnp.concatenate([in_degree, pad])
    outd_p = jnp.concatenate([out_degree, pad])
    return _centrality(ind_p, outd_p, itab, otab)


# speedup vs baseline: 2.9222x; 1.3704x over previous
import functools

import jax
import jax.numpy as jnp
from jax import lax
from jax.experimental import pallas as pl
from jax.experimental.pallas import tpu as pltpu
from jax.experimental.pallas import tpu_sc as plsc

N = 50000
D = 256
MAX_DEGREE = 512
V = MAX_DEGREE + 1
NC = 2
NS = 16
NW = NC * NS
CHUNK = 112
ROWS_PER_W = 1568
CHUNKS_PER_W = ROWS_PER_W // CHUNK
TAIL_W = NW - 1
TAIL_CHUNKS = 13


def _clip_remap(idx_ref):
    def body(j, _):
        v = idx_ref[pl.ds(j * 16, 16)]
        v = jnp.minimum(jnp.maximum(v, 0), MAX_DEGREE)
        idx_ref[pl.ds(j * 16, 16)] = jnp.where(v == 0, V, v)
        return 0
    lax.fori_loop(0, CHUNK // 16, body, 0, unroll=True)


def _sc_body(ind_hbm, outd_hbm, itab_hbm, otab_hbm, out_hbm,
             idx_a0, idx_b0, rows_a0, rows_b0,
             idx_a1, idx_b1, rows_a1, rows_b1,
             sem_ga0, sem_gb0, sem_st0, sem_ga1, sem_gb1, sem_st1):
    bufs = ((idx_a0, idx_b0, rows_a0, rows_b0, sem_ga0, sem_gb0, sem_st0),
            (idx_a1, idx_b1, rows_a1, rows_b1, sem_ga1, sem_gb1, sem_st1))
    wid = lax.axis_index("s") * NC + lax.axis_index("c")
    w0 = wid * ROWS_PER_W
    is_tail = wid == TAIL_W
    nchunks = jnp.where(is_tail, TAIL_CHUNKS, CHUNKS_PER_W)

    def chunk_base(c):
        return jnp.where(jnp.logical_and(is_tail, c == TAIL_CHUNKS - 1),
                         N - CHUNK, w0 + c * CHUNK)

    def start(c, buf):
        idx_a, idx_b, rows_a, rows_b, sga, sgb, _ = buf
        base = chunk_base(c)
        pltpu.sync_copy(ind_hbm.at[pl.ds(base, CHUNK)], idx_a)
        pltpu.sync_copy(outd_hbm.at[pl.ds(base, CHUNK)], idx_b)
        _clip_remap(idx_a)
        _clip_remap(idx_b)
        pltpu.async_copy(itab_hbm.at[idx_a], rows_a, sga)
        pltpu.async_copy(otab_hbm.at[idx_b], rows_b, sgb)

    def wait_gathers(buf):
        idx_a, idx_b, rows_a, rows_b, sga, sgb, _ = buf
        pltpu.make_async_copy(itab_hbm.at[idx_a], rows_a, sga).wait()
        pltpu.make_async_copy(otab_hbm.at[idx_b], rows_b, sgb).wait()

    def wait_store(buf):
        rows_a, sst = buf[2], buf[6]
        pltpu.make_async_copy(rows_a, out_hbm.at[pl.ds(0, CHUNK)], sst).wait()

    def add_and_store(c, buf):
        _, _, rows_a, rows_b, _, _, sst = buf

        def add_row(r, _):
            for j in range(D // 16):
                plsc.addupdate(rows_a.at[r, pl.ds(j * 16, 16)],
                               rows_b[r, pl.ds(j * 16, 16)])
            return 0
        lax.fori_loop(0, CHUNK, add_row, 0)
        pltpu.async_copy(rows_a, out_hbm.at[pl.ds(chunk_base(c), CHUNK)], sst)

    start(0, bufs[0])

    def loop_body(c, _):
        def one_iter(p):
            cur, nxt = bufs[p], bufs[1 - p]

            @pl.when(c + 1 < nchunks)
            def _():
                @pl.when(c >= 1)
                def _():
                    wait_store(nxt)
                start(c + 1, nxt)

            wait_gathers(cur)
            add_and_store(c, cur)

        @pl.when(c % 2 == 0)
        def _():
            one_iter(0)

        @pl.when(c % 2 == 1)
        def _():
            one_iter(1)
        return 0

    lax.fori_loop(0, nchunks, loop_body, 0)
    wait_store(bufs[0])
    wait_store(bufs[1])


@functools.partial(jax.jit, donate_argnums=())
def _centrality(ind, outd, itab, otab):
    mesh = plsc.VectorSubcoreMesh(core_axis_name="c", subcore_axis_name="s",
                                  num_cores=NC, num_subcores=NS)
    return pl.kernel(
        _sc_body,
        out_type=jax.ShapeDtypeStruct((N, D), jnp.float32),
        mesh=mesh,
        scratch_types=[
            pltpu.VMEM((CHUNK,), jnp.int32),
            pltpu.VMEM((CHUNK,), jnp.int32),
            pltpu.VMEM((CHUNK, D), jnp.float32),
            pltpu.VMEM((CHUNK, D), jnp.float32),
            pltpu.VMEM((CHUNK,), jnp.int32),
            pltpu.VMEM((CHUNK,), jnp.int32),
            pltpu.VMEM((CHUNK, D), jnp.float32),
            pltpu.VMEM((CHUNK, D), jnp.float32),
            pltpu.SemaphoreType.DMA,
            pltpu.SemaphoreType.DMA,
            pltpu.SemaphoreType.DMA,
            pltpu.SemaphoreType.DMA,
            pltpu.SemaphoreType.DMA,
            pltpu.SemaphoreType.DMA,
        ],
    )(ind, outd, itab, otab)


def kernel(in_degree, out_degree, in_table, out_table):
    zero_row = jnp.zeros((1, D), jnp.float32)
    itab = jnp.concatenate([in_table, zero_row], axis=0)
    otab = jnp.concatenate([out_table, zero_row], axis=0)
    return _centrality(in_degree, out_degree, itab, otab)
